# Initial kernel scaffold; baseline (speedup 1.0000x reference)
import jax
import jax.numpy as jnp
from jax.experimental import pallas as pl


def kernel(x, embed_weight):
    raise NotImplementedError("write your pallas kernel here")


# speedup vs baseline: 1.0355x; 1.0355x over previous
import functools

import jax
import jax.numpy as jnp
from jax import lax
from jax.experimental import pallas as pl
from jax.experimental.pallas import tpu as pltpu
from jax.experimental.pallas import tpu_sc as plsc

_N = 8192
_K = 8192
_D = 256
_TM = 1024
_TK = 2048
_NT = _N // _TM
_KT = _K // _TK
_BETA = 0.25


def _dist_argmin_body(z_ref, et_ref, idx_ref, run_ref, pick_ref, loss_ref):
    t = pl.program_id(0)
    k = pl.program_id(1)

    z_blk = z_ref[...]
    et_blk = et_ref[...]

    scores = jnp.dot(z_blk, et_blk)
    zsq = jnp.sum(z_blk * z_blk, axis=1, keepdims=True)
    esq = jnp.sum(et_blk * et_blk, axis=0, keepdims=True)
    dist = (zsq + esq) - 2.0 * scores

    m = jnp.min(dist, axis=1)
    ids = lax.broadcasted_iota(jnp.int32, (_TM, _TK), 1)
    am = jnp.min(jnp.where(dist == m[:, None], ids, jnp.int32(2**30)),
                 axis=1) + k * _TK
    mb = m.astype(jnp.bfloat16).astype(jnp.float32)

    @pl.when(k == 0)
    def _init():
        run_ref[0, 0, :] = mb
        idx_ref[0, 0, :] = am
        pick_ref[0, 0, :] = m

    @pl.when(k > 0)
    def _update():
        r = run_ref[0, 0, :]
        lt = m < r
        take = lt | ((m == r) & (am < idx_ref[0, 0, :]))
        idx_ref[0, 0, :] = jnp.where(take, am, idx_ref[0, 0, :])
        pick_ref[0, 0, :] = jnp.where(take, m, pick_ref[0, 0, :])
        run_ref[0, 0, :] = jnp.where(lt, mb, r)

    @pl.when((t == 0) & (k == 0))
    def _zero_loss():
        loss_ref[0, 0] = 0.0

    @pl.when(k == _KT - 1)
    def _acc_loss():
        loss_ref[0, 0] += jnp.sum(pick_ref[0, 0, :])

    @pl.when((t == _NT - 1) & (k == _KT - 1))
    def _finish_loss():
        loss_ref[0, 0] = loss_ref[0, 0] * ((1.0 + _BETA) / (_N * _D))


def _dist_argmin(z2d, et):
    idx3, _run, _pick, loss = pl.pallas_call(
        _dist_argmin_body,
        grid=(_NT, _KT),
        in_specs=[
            pl.BlockSpec((_TM, _D), lambda t, k: (t, 0)),
            pl.BlockSpec((_D, _TK), lambda t, k: (0, k)),
        ],
        out_specs=[
            pl.BlockSpec((1, 1, _TM), lambda t, k: (t, 0, 0)),
            pl.BlockSpec((1, 1, _TM), lambda t, k: (t, 0, 0)),
            pl.BlockSpec((1, 1, _TM), lambda t, k: (t, 0, 0)),
            pl.BlockSpec((1, 1), lambda t, k: (0, 0),
                         memory_space=pltpu.SMEM),
        ],
        out_shape=[
            jax.ShapeDtypeStruct((_NT, 1, _TM), jnp.int32),
            jax.ShapeDtypeStruct((_NT, 1, _TM), jnp.float32),
            jax.ShapeDtypeStruct((_NT, 1, _TM), jnp.float32),
            jax.ShapeDtypeStruct((1, 1), jnp.float32),
        ],
    )(z2d, et)
    return idx3.reshape(_N), loss.reshape(())


_NC = 2
_NS = 16
_NW = _NC * _NS
_BPW = _N // _NW


def _sc_gather_body(table_hbm, idx_hbm, out_hbm, idx_v, rows_v, sem):
    wid = lax.axis_index("s") * _NC + lax.axis_index("c")
    base = wid * _BPW
    pltpu.sync_copy(idx_hbm.at[pl.ds(base, _BPW)], idx_v)
    pltpu.async_copy(table_hbm.at[idx_v], rows_v, sem).wait()
    pltpu.sync_copy(rows_v, out_hbm.at[pl.ds(base, _BPW)])


@functools.cache
def _sc_gather():
    return pl.kernel(
        _sc_gather_body,
        mesh=plsc.VectorSubcoreMesh(core_axis_name="c", subcore_axis_name="s"),
        out_type=jax.ShapeDtypeStruct((_N, _D), jnp.float32),
        scratch_types=[
            pltpu.VMEM((_BPW,), jnp.int32),
            pltpu.VMEM((_BPW, _D), jnp.float32),
            pltpu.SemaphoreType.DMA,
        ],
    )


def kernel(x, embed_weight):
    z2d = jnp.transpose(x, (0, 2, 3, 1)).reshape(_N, _D)
    et = embed_weight.T
    idx, loss = _dist_argmin(z2d, et)
    zq = _sc_gather()(embed_weight, idx)
    zq = jnp.transpose(zq.reshape(8, 32, 32, _D), (0, 3, 1, 2))
    return (zq, loss)


# speedup vs baseline: 1.1738x; 1.1336x over previous
import functools

import jax
import jax.numpy as jnp
from jax import lax
from jax.experimental import pallas as pl
from jax.experimental.pallas import tpu as pltpu
from jax.experimental.pallas import tpu_sc as plsc

_N = 8192
_K = 8192
_D = 256
_TM = 1024
_TK = 2048
_NT = _N // _TM
_KT = _K // _TK
_BETA = 0.25


def _dist_argmin_body(z_ref, et2_ref, esq_ref, idx_ref, loss_ref,
                      run_s, pick_s, amin_s):
    t = pl.program_id(0)
    k = pl.program_id(1)

    z_blk = z_ref[...]
    scores2 = jnp.dot(z_blk, et2_ref[...])
    zsq = jnp.sum(z_blk * z_blk, axis=1, keepdims=True)
    dist = (zsq + esq_ref[...]) - scores2

    m = jnp.min(dist, axis=1, keepdims=True)
    ids = lax.broadcasted_iota(jnp.int32, (_TM, _TK), 1)
    am = jnp.min(jnp.where(dist == m, ids, jnp.int32(2**30)),
                 axis=1, keepdims=True) + k * _TK
    mb = m.astype(jnp.bfloat16).astype(jnp.float32)

    @pl.when(k == 0)
    def _init():
        run_s[...] = mb
        amin_s[...] = am
        pick_s[...] = m

    @pl.when(k > 0)
    def _update():
        r = run_s[...]
        lt = m < r
        take = lt | ((m == r) & (am < amin_s[...]))
        amin_s[...] = jnp.where(take, am, amin_s[...])
        pick_s[...] = jnp.where(take, m, pick_s[...])
        run_s[...] = jnp.where(lt, mb, r)

    @pl.when((t == 0) & (k == 0))
    def _zero_loss():
        loss_ref[0, 0] = 0.0

    @pl.when(k == _KT - 1)
    def _final():
        idx_ref[0, 0, :] = amin_s[...][:, 0]
        loss_ref[0, 0] += jnp.sum(pick_s[...])

    @pl.when((t == _NT - 1) & (k == _KT - 1))
    def _finish_loss():
        loss_ref[0, 0] = loss_ref[0, 0] * ((1.0 + _BETA) / (_N * _D))


def _dist_argmin(z2d, et2, esq2d):
    idx3, loss = pl.pallas_call(
        _dist_argmin_body,
        grid=(_NT, _KT),
        in_specs=[
            pl.BlockSpec((_TM, _D), lambda t, k: (t, 0)),
            pl.BlockSpec((_D, _TK), lambda t, k: (0, k)),
            pl.BlockSpec((1, _TK), lambda t, k: (0, k)),
        ],
        out_specs=[
            pl.BlockSpec((1, 1, _TM), lambda t, k: (t, 0, 0)),
            pl.BlockSpec((1, 1), lambda t, k: (0, 0),
                         memory_space=pltpu.SMEM),
        ],
        out_shape=[
            jax.ShapeDtypeStruct((_NT, 1, _TM), jnp.int32),
            jax.ShapeDtypeStruct((1, 1), jnp.float32),
        ],
        scratch_shapes=[
            pltpu.VMEM((_TM, 1), jnp.float32),
            pltpu.VMEM((_TM, 1), jnp.float32),
            pltpu.VMEM((_TM, 1), jnp.int32),
        ],
    )(z2d, et2, esq2d)
    return idx3.reshape(_N), loss.reshape(())


_NC = 2
_NS = 16
_NW = _NC * _NS
_BPW = _N // _NW


def _sc_gather_body(table_hbm, idx_hbm, out_hbm, idx_v, rows_v, sem):
    wid = lax.axis_index("s") * _NC + lax.axis_index("c")
    base = wid * _BPW
    pltpu.sync_copy(idx_hbm.at[pl.ds(base, _BPW)], idx_v)
    pltpu.async_copy(table_hbm.at[idx_v], rows_v, sem).wait()
    pltpu.sync_copy(rows_v, out_hbm.at[pl.ds(base, _BPW)])


@functools.cache
def _sc_gather():
    return pl.kernel(
        _sc_gather_body,
        mesh=plsc.VectorSubcoreMesh(core_axis_name="c", subcore_axis_name="s"),
        out_type=jax.ShapeDtypeStruct((_N, _D), jnp.float32),
        scratch_types=[
            pltpu.VMEM((_BPW,), jnp.int32),
            pltpu.VMEM((_BPW, _D), jnp.float32),
            pltpu.SemaphoreType.DMA,
        ],
    )


def kernel(x, embed_weight):
    z2d = jnp.transpose(x, (0, 2, 3, 1)).reshape(_N, _D)
    et2 = (embed_weight + embed_weight).T
    esq2d = jnp.sum(embed_weight * embed_weight,
                    axis=1)[None, :]
    idx, loss = _dist_argmin(z2d, et2, esq2d)
    zq = _sc_gather()(embed_weight, idx)
    zq = jnp.transpose(zq.reshape(8, 32, 32, _D), (0, 3, 1, 2))
    return (zq, loss)


# speedup vs baseline: 1.2908x; 1.0997x over previous
import functools

import jax
import jax.numpy as jnp
from jax import lax
from jax.experimental import pallas as pl
from jax.experimental.pallas import tpu as pltpu
from jax.experimental.pallas import tpu_sc as plsc

_N = 8192
_K = 8192
_D = 256
_TM = 1024
_TK = 2048
_NT = _N // _TM
_KT = _K // _TK
_BETA = 0.25


def _dist_argmin_body(zt_ref, e2_ref, esq_ref, idx_ref, loss_ref,
                      run_s, pick_s, amin_s):
    t = pl.program_id(0)
    k = pl.program_id(1)

    zt_blk = zt_ref[0]
    scores2 = jnp.dot(e2_ref[...], zt_blk)
    zsq = jnp.sum(zt_blk * zt_blk, axis=0, keepdims=True)
    dist = (zsq + esq_ref[...]) - scores2

    m = jnp.min(dist, axis=0, keepdims=True)
    ids = lax.broadcasted_iota(jnp.int32, (_TK, _TM), 0)
    am = jnp.min(jnp.where(dist == m, ids, jnp.int32(2**30)),
                 axis=0, keepdims=True) + k * _TK
    mb = m.astype(jnp.bfloat16).astype(jnp.float32)

    @pl.when(k == 0)
    def _init():
        run_s[...] = mb
        amin_s[...] = am
        pick_s[...] = m

    @pl.when(k > 0)
    def _update():
        r = run_s[...]
        lt = m < r
        take = lt | ((m == r) & (am < amin_s[...]))
        amin_s[...] = jnp.where(take, am, amin_s[...])
        pick_s[...] = jnp.where(take, m, pick_s[...])
        run_s[...] = jnp.where(lt, mb, r)

    @pl.when((t == 0) & (k == 0))
    def _zero_loss():
        loss_ref[0, 0] = 0.0

    @pl.when(k == _KT - 1)
    def _final():
        idx_ref[0, 0, :] = amin_s[0, :]
        loss_ref[0, 0] += jnp.sum(pick_s[...])

    @pl.when((t == _NT - 1) & (k == _KT - 1))
    def _finish_loss():
        loss_ref[0, 0] = loss_ref[0, 0] * ((1.0 + _BETA) / (_N * _D))


def _dist_argmin(zt3, e2, esq2d):
    idx3, loss = pl.pallas_call(
        _dist_argmin_body,
        grid=(_NT, _KT),
        in_specs=[
            pl.BlockSpec((1, _D, _TM), lambda t, k: (t, 0, 0)),
            pl.BlockSpec((_TK, _D), lambda t, k: (k, 0)),
            pl.BlockSpec((_TK, 1), lambda t, k: (k, 0)),
        ],
        out_specs=[
            pl.BlockSpec((1, 1, _TM), lambda t, k: (t, 0, 0)),
            pl.BlockSpec((1, 1), lambda t, k: (0, 0),
                         memory_space=pltpu.SMEM),
        ],
        out_shape=[
            jax.ShapeDtypeStruct((_NT, 1, _TM), jnp.int32),
            jax.ShapeDtypeStruct((1, 1), jnp.float32),
        ],
        scratch_shapes=[
            pltpu.VMEM((1, _TM), jnp.float32),
            pltpu.VMEM((1, _TM), jnp.float32),
            pltpu.VMEM((1, _TM), jnp.int32),
        ],
    )(zt3, e2, esq2d)
    return idx3.reshape(_N), loss.reshape(())


_NC = 2
_NS = 16
_NW = _NC * _NS
_BPW = _N // _NW


def _sc_gather_body(table_hbm, idx_hbm, out_hbm, idx_v, rows_v, sem):
    wid = lax.axis_index("s") * _NC + lax.axis_index("c")
    base = wid * _BPW
    pltpu.sync_copy(idx_hbm.at[pl.ds(base, _BPW)], idx_v)
    pltpu.async_copy(table_hbm.at[idx_v], rows_v, sem).wait()
    pltpu.sync_copy(rows_v, out_hbm.at[pl.ds(base, _BPW)])


@functools.cache
def _sc_gather():
    return pl.kernel(
        _sc_gather_body,
        mesh=plsc.VectorSubcoreMesh(core_axis_name="c", subcore_axis_name="s"),
        out_type=jax.ShapeDtypeStruct((_N, _D), jnp.float32),
        scratch_types=[
            pltpu.VMEM((_BPW,), jnp.int32),
            pltpu.VMEM((_BPW, _D), jnp.float32),
            pltpu.SemaphoreType.DMA,
        ],
    )


def kernel(x, embed_weight):
    zt3 = x.reshape(_NT, _D, _TM)
    e2 = embed_weight + embed_weight
    esq2d = jnp.sum(embed_weight * embed_weight,
                    axis=1)[:, None]
    idx, loss = _dist_argmin(zt3, e2, esq2d)
    zq = _sc_gather()(embed_weight, idx)
    zq = jnp.transpose(zq.reshape(8, 32, 32, _D), (0, 3, 1, 2))
    return (zq, loss)


# speedup vs baseline: 1.3480x; 1.0443x over previous
import functools

import jax
import jax.numpy as jnp
from jax import lax
from jax.experimental import pallas as pl
from jax.experimental.pallas import tpu as pltpu
from jax.experimental.pallas import tpu_sc as plsc

_N = 8192
_K = 8192
_D = 256
_TM = 1024
_TK = 2048
_NT = _N // _TM
_KT = _K // _TK
_BETA = 0.25


def _dist_argmin_body(zt_ref, e_ref, idx_ref, loss_ref,
                      run_s, pick_s, amin_s, esq_s):
    t = pl.program_id(0)
    k = pl.program_id(1)

    e_blk = e_ref[...]

    @pl.when(t == 0)
    def _strip_esq():
        esq_s[pl.ds(k * _TK, _TK), :] = jnp.sum(e_blk * e_blk, axis=1,
                                                keepdims=True)

    zt_blk = zt_ref[0]
    scores2 = jnp.dot(e_blk, zt_blk + zt_blk)
    zsq = jnp.sum(zt_blk * zt_blk, axis=0, keepdims=True)
    esq = esq_s[pl.ds(k * _TK, _TK), :]
    dist = (zsq + esq) - scores2

    m = jnp.min(dist, axis=0, keepdims=True)
    ids = lax.broadcasted_iota(jnp.int32, (_TK, _TM), 0)
    am = jnp.min(jnp.where(dist == m, ids, jnp.int32(2**30)),
                 axis=0, keepdims=True) + k * _TK
    mb = m.astype(jnp.bfloat16).astype(jnp.float32)

    @pl.when(k == 0)
    def _init():
        run_s[...] = mb
        amin_s[...] = am
        pick_s[...] = m

    @pl.when(k > 0)
    def _update():
        r = run_s[...]
        lt = m < r
        take = lt | ((m == r) & (am < amin_s[...]))
        amin_s[...] = jnp.where(take, am, amin_s[...])
        pick_s[...] = jnp.where(take, m, pick_s[...])
        run_s[...] = jnp.where(lt, mb, r)

    @pl.when((t == 0) & (k == 0))
    def _zero_loss():
        loss_ref[0, 0] = 0.0

    @pl.when(k == _KT - 1)
    def _final():
        idx_ref[0, 0, :] = amin_s[0, :]
        loss_ref[0, 0] += jnp.sum(pick_s[...])

    @pl.when((t == _NT - 1) & (k == _KT - 1))
    def _finish_loss():
        loss_ref[0, 0] = loss_ref[0, 0] * ((1.0 + _BETA) / (_N * _D))


def _dist_argmin(zt3, e):
    idx3, loss = pl.pallas_call(
        _dist_argmin_body,
        grid=(_NT, _KT),
        in_specs=[
            pl.BlockSpec((1, _D, _TM), lambda t, k: (t, 0, 0)),
            pl.BlockSpec((_TK, _D), lambda t, k: (k, 0)),
        ],
        out_specs=[
            pl.BlockSpec((1, 1, _TM), lambda t, k: (t, 0, 0)),
            pl.BlockSpec((1, 1), lambda t, k: (0, 0),
                         memory_space=pltpu.SMEM),
        ],
        out_shape=[
            jax.ShapeDtypeStruct((_NT, 1, _TM), jnp.int32),
            jax.ShapeDtypeStruct((1, 1), jnp.float32),
        ],
        scratch_shapes=[
            pltpu.VMEM((1, _TM), jnp.float32),
            pltpu.VMEM((1, _TM), jnp.float32),
            pltpu.VMEM((1, _TM), jnp.int32),
            pltpu.VMEM((_K, 1), jnp.float32),
        ],
    )(zt3, e)
    return idx3.reshape(_N), loss.reshape(())


_NC = 2
_NS = 16
_NW = _NC * _NS
_BPW = _N // _NW


def _sc_gather_body(table_hbm, idx_hbm, out_hbm, idx_v, rows_v, sem):
    wid = lax.axis_index("s") * _NC + lax.axis_index("c")
    base = wid * _BPW
    pltpu.sync_copy(idx_hbm.at[pl.ds(base, _BPW)], idx_v)
    pltpu.async_copy(table_hbm.at[idx_v], rows_v, sem).wait()
    pltpu.sync_copy(rows_v, out_hbm.at[pl.ds(base, _BPW)])


@functools.cache
def _sc_gather():
    return pl.kernel(
        _sc_gather_body,
        mesh=plsc.VectorSubcoreMesh(core_axis_name="c", subcore_axis_name="s"),
        out_type=jax.ShapeDtypeStruct((_N, _D), jnp.float32),
        scratch_types=[
            pltpu.VMEM((_BPW,), jnp.int32),
            pltpu.VMEM((_BPW, _D), jnp.float32),
            pltpu.SemaphoreType.DMA,
        ],
    )


def kernel(x, embed_weight):
    zt3 = x.reshape(_NT, _D, _TM)
    idx, loss = _dist_argmin(zt3, embed_weight)
    zq = _sc_gather()(embed_weight, idx)
    zq = jnp.transpose(zq.reshape(8, 32, 32, _D), (0, 3, 1, 2))
    return (zq, loss)
